# baseline (speedup 1.0000x reference)
import functools

import jax
import jax.numpy as jnp
from jax import lax
from jax.experimental import pallas as pl
from jax.experimental.pallas import tpu as pltpu
from jax.experimental.pallas import tpu_sc as plsc

_N = 1024
_NMASK = 768
_TAIL = _N - _NMASK
_CAP = 9
_G = 16
_NGRP = _N // _G


def _threefry_bits(f):
    ks0 = jnp.uint32(0)
    ks1 = jnp.uint32(42)
    ks2 = jnp.uint32(0x1BD11BDA) ^ ks0 ^ ks1
    ks = (ks0, ks1, ks2)
    rot = ((13, 15, 26, 6), (17, 29, 16, 24))
    x0 = jnp.broadcast_to(ks0, f.shape)
    x1 = f + ks1
    for i in range(5):
        for r in rot[i % 2]:
            x0 = x0 + x1
            x1 = (x1 << r) | (x1 >> (32 - r))
            x1 = x0 ^ x1
        x0 = x0 + ks[(i + 1) % 3]
        x1 = x1 + ks[(i + 2) % 3] + jnp.uint32(i + 1)
    return x0 ^ x1


_TCB = 4


def _mant_body(out_ref, *, grp0):
    g = pl.program_id(0)
    q = lax.broadcasted_iota(jnp.int32, (_TCB, 128, 128), 0)
    a = lax.broadcasted_iota(jnp.int32, (_TCB, 128, 128), 1)
    b = lax.broadcasted_iota(jnp.int32, (_TCB, 128, 128), 2)
    j = a * 8 + (b >> 4)
    i = (grp0 + g * _TCB + q) * _G + (b & 15)
    f = (i * _N + j).astype(jnp.uint32)
    out_ref[...] = (_threefry_bits(f) >> 9).astype(jnp.int32)


_sc_mesh = plsc.VectorSubcoreMesh(core_axis_name="c", subcore_axis_name="s")


@functools.partial(
    pl.kernel,
    out_type=jax.ShapeDtypeStruct((_N // 2, _N), jnp.int32),
    mesh=_sc_mesh,
    scratch_types=[
        pltpu.VMEM((128, 128), jnp.int32),
        pltpu.VMEM((_N * _G,), jnp.int32),
        pltpu.VMEM((_N * _G,), jnp.int32),
        pltpu.VMEM((_N * _G,), jnp.int32),
        pltpu.VMEM((_N * _G,), jnp.int32),
        pltpu.VMEM((_G, _N), jnp.int32),
    ],
    compiler_params=pltpu.CompilerParams(needs_layout_passes=False),
)
def _sc_mask(mant_hbm, out_hbm, mant_v, ha_v, hb_v, hist_v, s_v, out_v):
    wid = lax.axis_index("s") * 2 + lax.axis_index("c")
    lanes = lax.iota(jnp.int32, 16)
    ones = jnp.ones((_G,), jnp.int32)
    zeros = jnp.zeros((_G,), jnp.int32)
    _H = _N // 2

    if True:
        grp = wid
        base = grp * _G
        pltpu.sync_copy(mant_hbm.at[grp], mant_v)

        @plsc.parallel_loop(0, _N, unroll=8)
        def _(b):
            ha_v[pl.ds(b * _G, _G)] = zeros
            hb_v[pl.ds(b * _G, _G)] = zeros

        @plsc.parallel_loop(0, _H, unroll=8)
        def _(j):
            addr_a = ((mant_v[j >> 3, pl.ds((j & 7) * _G, _G)] >> 13) << 4) + lanes
            plsc.addupdate_scatter(ha_v, [addr_a], ones)
            jb = j + _H
            addr_b = ((mant_v[jb >> 3, pl.ds((jb & 7) * _G, _G)] >> 13) << 4) + lanes
            plsc.addupdate_scatter(hb_v, [addr_b], ones)

        @plsc.parallel_loop(0, _N, unroll=4, carry=zeros)
        def _(b, acc):
            sl = pl.ds(b * _G, _G)
            ca = ha_v[sl]
            cb = hb_v[sl]
            tot = ca + cb
            hist_v[sl] = tot
            ha_v[sl] = acc
            hb_v[sl] = acc + ca
            return acc + tot

        def place_body(j, _):
            ma = mant_v[j >> 3, pl.ds((j & 7) * _G, _G)]
            addr_a = ((ma >> 13) << 4) + lanes
            k2a = ((ma & 0x1FFF) << 10) | j
            cur_a = plsc.load_gather(ha_v, [addr_a])
            plsc.store_scatter(s_v, [(cur_a << 4) + lanes], k2a)
            plsc.addupdate_scatter(ha_v, [addr_a], ones)
            jb = j + _H
            mb = mant_v[jb >> 3, pl.ds((jb & 7) * _G, _G)]
            addr_b = ((mb >> 13) << 4) + lanes
            k2b = ((mb & 0x1FFF) << 10) | jb
            cur_b = plsc.load_gather(hb_v, [addr_b])
            plsc.store_scatter(s_v, [(cur_b << 4) + lanes], k2b)
            plsc.addupdate_scatter(hb_v, [addr_b], ones)
            return 0

        lax.fori_loop(0, _H, place_body, 0, unroll=4)

        for r in range(_G):

            @plsc.parallel_loop(0, _N // _G, unroll=8)
            def _(c, r=r):
                out_v[r, pl.ds(c * _G, _G)] = ones

        @plsc.parallel_loop(0, _TAIL, unroll=2)
        def _(t):
            j = t + _NMASK
            m = mant_v[j >> 3, pl.ds((j & 7) * _G, _G)]
            addr = ((m >> 13) << 4) + lanes
            k2k = ((m & 0x1FFF) << 10) | j
            endc = plsc.load_gather(hb_v, [addr])
            cnt = plsc.load_gather(hist_v, [addr])
            start = endc - cnt
            start16 = (start << 4) + lanes
            fine = zeros
            for c in range(_CAP):
                msk = cnt > c
                occ = plsc.load_gather(s_v, [start16 + c * _G], mask=msk)
                fine = fine + jnp.where(msk & (occ < k2k), 1, 0)
            plsc.store_scatter(out_v, [lanes, start + fine], zeros)

        pltpu.sync_copy(out_v, out_hbm.at[pl.ds(base, _G), :])


def kernel(x):
    del x
    halves = []
    for h in range(2):
        mant_h = pl.pallas_call(
            functools.partial(_mant_body, grp0=h * (_NGRP // 2)),
            grid=(_NGRP // 2 // _TCB,),
            out_shape=jax.ShapeDtypeStruct((_NGRP // 2, 128, 128), jnp.int32),
            out_specs=pl.BlockSpec((_TCB, 128, 128), lambda g: (g, 0, 0)),
        )()
        halves.append(_sc_mask(mant_h).astype(bool))
    return jnp.concatenate(halves, axis=0)
